# baseline (speedup 1.0000x reference)
import functools
import math

import jax
import jax.numpy as jnp
from jax import lax
from jax.experimental import pallas as pl
from jax.experimental.pallas import tpu as pltpu
from jax.experimental.pallas import tpu_sc as plsc

N_EMBD = 2048
NUM_EXPERTS = 4
BT = 8192
HID = 4 * N_EMBD
RB = 16
RBLK = BT // RB
MBLK = 1024
MB = BT // MBLK
NBLK = MB + NUM_EXPERTS
NPAD = NBLK * MBLK
H_TILES = 16
H_CHUNK = HID // H_TILES


def _router_body(x_ref, rw_ref, e_ref, rank_ref, counts_ref, sump_ref, hz_ref):
    i = pl.program_id(0)
    x = x_ref[...]
    logits = lax.dot_general(x, rw_ref[...], (((1,), (1,)), ((), ())),
                             preferred_element_type=jnp.float32)
    m = jnp.max(logits, axis=-1, keepdims=True)
    ex = jnp.exp(logits - m)
    s = jnp.sum(ex, axis=-1, keepdims=True)
    probs = ex / s
    lse = m + jnp.log(s)

    e = jnp.argmax(logits, axis=-1).astype(jnp.int32)
    eids = lax.broadcasted_iota(jnp.int32, (RBLK, NUM_EXPERTS), 1)
    onehot = (eids == e[:, None]).astype(jnp.float32)

    r_i = lax.broadcasted_iota(jnp.int32, (RBLK, RBLK), 0)
    c_i = lax.broadcasted_iota(jnp.int32, (RBLK, RBLK), 1)
    tril = (c_i < r_i).astype(jnp.float32)
    excl = lax.dot_general(tril, onehot, (((1,), (0,)), ((), ())),
                           preferred_element_type=jnp.float32)

    @pl.when(i == 0)
    def _():
        counts_ref[...] = jnp.zeros_like(counts_ref)
        sump_ref[...] = jnp.zeros_like(sump_ref)
        hz_ref[...] = jnp.zeros_like(hz_ref)

    carry = counts_ref[0, :NUM_EXPERTS]
    rank = jnp.sum(onehot * (excl + carry[None, :]), axis=-1)
    e_ref[...] = e
    rank_ref[...] = rank.astype(jnp.int32)

    blk_counts = jnp.sum(onehot, axis=0)
    counts_ref[0, :NUM_EXPERTS] += blk_counts
    sump_ref[0, :NUM_EXPERTS] += jnp.sum(probs, axis=0)
    token_h = -jnp.sum(probs * jnp.log(probs + 1e-9), axis=-1)
    lane = lax.broadcasted_iota(jnp.int32, (1, 16), 1)
    hz_ref[...] += (jnp.where(lane == 0, jnp.sum(token_h), 0.0)
                    + jnp.where(lane == 1, jnp.sum(lse[:, 0] ** 2), 0.0))


def _router(x_flat, router_w):
    return pl.pallas_call(
        _router_body,
        grid=(RB,),
        in_specs=[
            pl.BlockSpec((RBLK, N_EMBD), lambda i: (i, 0)),
            pl.BlockSpec((NUM_EXPERTS, N_EMBD), lambda i: (0, 0)),
        ],
        out_specs=[
            pl.BlockSpec((RBLK,), lambda i: (i,)),
            pl.BlockSpec((RBLK,), lambda i: (i,)),
            pl.BlockSpec((1, 16), lambda i: (0, 0)),
            pl.BlockSpec((1, 16), lambda i: (0, 0)),
            pl.BlockSpec((1, 16), lambda i: (0, 0)),
        ],
        out_shape=[
            jax.ShapeDtypeStruct((BT,), jnp.int32),
            jax.ShapeDtypeStruct((BT,), jnp.int32),
            jax.ShapeDtypeStruct((1, 16), jnp.float32),
            jax.ShapeDtypeStruct((1, 16), jnp.float32),
            jax.ShapeDtypeStruct((1, 16), jnp.float32),
        ],
    )(x_flat, router_w)


def _dest_body(po_ref, e_ref, rank_ref, dest_ref):
    e = e_ref[...]
    dest = rank_ref[...]
    for k in range(NUM_EXPERTS):
        dest = dest + jnp.where(e == k, po_ref[k], 0)
    dest_ref[...] = dest


def _dest(po16, e, rank):
    return pl.pallas_call(
        _dest_body,
        grid=(RB,),
        in_specs=[
            pl.BlockSpec(memory_space=pltpu.SMEM),
            pl.BlockSpec((RBLK,), lambda i: (i,)),
            pl.BlockSpec((RBLK,), lambda i: (i,)),
        ],
        out_specs=pl.BlockSpec((RBLK,), lambda i: (i,)),
        out_shape=jax.ShapeDtypeStruct((BT,), jnp.int32),
    )(po16, e, rank)


_NW = 32
_TPW = BT // _NW
_ROWS = 32
_NCH = _TPW // _ROWS


def _dispatch_body(dest_hbm, x_hbm, xs_hbm, dest_v, idx_v, rows_v):
    wid = lax.axis_index("s") * 2 + lax.axis_index("c")
    base = wid * _TPW
    pltpu.sync_copy(dest_hbm.at[pl.ds(base, _TPW)], dest_v)
    for c in range(_NCH):
        idx_v[pl.ds(0, 16)] = dest_v[pl.ds(c * _ROWS, 16)]
        idx_v[pl.ds(16, 16)] = dest_v[pl.ds(c * _ROWS + 16, 16)]
        pltpu.sync_copy(x_hbm.at[pl.ds(base + c * _ROWS, _ROWS)], rows_v)
        pltpu.sync_copy(rows_v, xs_hbm.at[idx_v])


def _combine_body(dest_hbm, ys_hbm, y_hbm, dest_v, idx_v, rows_v):
    wid = lax.axis_index("s") * 2 + lax.axis_index("c")
    base = wid * _TPW
    pltpu.sync_copy(dest_hbm.at[pl.ds(base, _TPW)], dest_v)
    for c in range(_NCH):
        idx_v[pl.ds(0, 16)] = dest_v[pl.ds(c * _ROWS, 16)]
        idx_v[pl.ds(16, 16)] = dest_v[pl.ds(c * _ROWS + 16, 16)]
        pltpu.sync_copy(ys_hbm.at[idx_v], rows_v)
        pltpu.sync_copy(rows_v, y_hbm.at[pl.ds(base + c * _ROWS, _ROWS)])


@functools.lru_cache(maxsize=None)
def _sc_kernels():
    mesh = plsc.VectorSubcoreMesh(core_axis_name="c", subcore_axis_name="s",
                                  num_cores=2, num_subcores=16)
    dispatch = pl.kernel(
        _dispatch_body,
        out_type=jax.ShapeDtypeStruct((NPAD, N_EMBD), jnp.float32),
        mesh=mesh,
        scratch_types=[
            pltpu.VMEM((_TPW,), jnp.int32),
            pltpu.VMEM((_ROWS,), jnp.int32),
            pltpu.VMEM((_ROWS, N_EMBD), jnp.float32),
        ],
    )
    combine = pl.kernel(
        _combine_body,
        out_type=jax.ShapeDtypeStruct((BT, N_EMBD), jnp.float32),
        mesh=mesh,
        scratch_types=[
            pltpu.VMEM((_TPW,), jnp.int32),
            pltpu.VMEM((_ROWS,), jnp.int32),
            pltpu.VMEM((_ROWS, N_EMBD), jnp.float32),
        ],
    )
    return dispatch, combine


def _mlp_body(meta_ref, xs_ref, wfc_ref, wpj_ref, yin_ref, out_ref):
    del yin_ref
    b = pl.program_id(0)
    h = pl.program_id(1)
    nb = meta_ref[1]

    @pl.when(b < nb)
    def _():
        x = xs_ref[...].astype(jnp.bfloat16)
        wfc = wfc_ref[...].astype(jnp.bfloat16)
        wpj = wpj_ref[...].astype(jnp.bfloat16)
        hact = lax.dot_general(x, wfc, (((1,), (1,)), ((), ())),
                               preferred_element_type=jnp.float32)
        hact = jnp.square(jnp.maximum(hact, 0.0)).astype(jnp.bfloat16)
        part = lax.dot_general(hact, wpj, (((1,), (1,)), ((), ())),
                               preferred_element_type=jnp.float32)

        @pl.when(h == 0)
        def _():
            out_ref[...] = part

        @pl.when(h != 0)
        def _():
            out_ref[...] += part


def _tok_idx(b, h, meta):
    return (meta[0] + jnp.minimum(b, meta[1] - 1), 0)


def _zig_idx(b, h, meta):
    nb = meta[1]
    zz = jnp.where(b % 2 == 0, h, H_TILES - 1 - h)
    frozen = jnp.where((nb - 1) % 2 == 0, H_TILES - 1, 0)
    return jnp.where(b < nb, zz, frozen)


def _expert_mlp(meta, xs, wfc, wpj, ys):
    grid_spec = pltpu.PrefetchScalarGridSpec(
        num_scalar_prefetch=1,
        grid=(MB, H_TILES),
        in_specs=[
            pl.BlockSpec((MBLK, N_EMBD), _tok_idx),
            pl.BlockSpec((H_CHUNK, N_EMBD),
                         lambda b, h, meta: (_zig_idx(b, h, meta), 0)),
            pl.BlockSpec((N_EMBD, H_CHUNK),
                         lambda b, h, meta: (0, _zig_idx(b, h, meta))),
            pl.BlockSpec((8, 128), lambda b, h, meta: (0, 0)),
        ],
        out_specs=pl.BlockSpec((MBLK, N_EMBD), _tok_idx),
    )
    return pl.pallas_call(
        _mlp_body,
        grid_spec=grid_spec,
        out_shape=jax.ShapeDtypeStruct((NPAD, N_EMBD), jnp.float32),
        input_output_aliases={4: 0},
        compiler_params=pltpu.CompilerParams(
            dimension_semantics=("arbitrary", "arbitrary"),
        ),
    )(meta, xs, wfc, wpj, ys)


def kernel(x, router_w, w_fc_0, w_proj_0, w_fc_1, w_proj_1,
           w_fc_2, w_proj_2, w_fc_3, w_proj_3):
    B, T, C = x.shape
    x_flat = x.reshape(BT, C)

    e, rank, counts_f, sump, hz = _router(x_flat, router_w)
    counts = counts_f[0, :NUM_EXPERTS]
    counts_i = counts.astype(jnp.int32)

    nbb = jnp.maximum((counts_i + (MBLK - 1)) // MBLK, 1)
    pob = jnp.concatenate(
        [jnp.zeros((1,), jnp.int32), jnp.cumsum(nbb).astype(jnp.int32)])
    po16 = jnp.zeros((16,), jnp.int32).at[:NUM_EXPERTS].set(pob[:NUM_EXPERTS]
                                                            * MBLK)
    dest = _dest(po16, e, rank)
    dispatch, combine = _sc_kernels()
    xs = dispatch(dest, x_flat)
    ys = jnp.zeros((NPAD, N_EMBD), jnp.float32)
    for ei, (wfc, wpj) in enumerate([(w_fc_0, w_proj_0), (w_fc_1, w_proj_1),
                                     (w_fc_2, w_proj_2), (w_fc_3, w_proj_3)]):
        meta = jnp.stack([pob[ei], nbb[ei]])
        ys = _expert_mlp(meta, xs, wfc, wpj, ys)
    y_flat = combine(dest, ys)

    inv_bt = 1.0 / float(BT)
    actual = counts * inv_bt
    expected = sump[0, :NUM_EXPERTS] * inv_bt
    aux = NUM_EXPERTS * jnp.sum(actual * expected)
    router_entropy = hz[0, 0] * inv_bt / math.log(float(NUM_EXPERTS))
    z_loss = hz[0, 1] * inv_bt
    return (y_flat.reshape(B, T, C), aux, z_loss, router_entropy, actual)


# speedup vs baseline: 1.0676x; 1.0676x over previous
import functools
import math

import jax
import jax.numpy as jnp
from jax import lax
from jax.experimental import pallas as pl
from jax.experimental.pallas import tpu as pltpu
from jax.experimental.pallas import tpu_sc as plsc

N_EMBD = 2048
NUM_EXPERTS = 4
BT = 8192
HID = 4 * N_EMBD
BLK = 512
NBLK = BT // BLK + NUM_EXPERTS
NPAD = NBLK * BLK
RB = BT // BLK
H_TILES = 8
H_CHUNK = HID // H_TILES


def _router_body(x_ref, rw_ref, e_ref, rank_ref, counts_ref, sump_ref, hz_ref):
    i = pl.program_id(0)
    x = x_ref[...]
    logits = lax.dot_general(x, rw_ref[...], (((1,), (1,)), ((), ())),
                             preferred_element_type=jnp.float32)
    m = jnp.max(logits, axis=-1, keepdims=True)
    ex = jnp.exp(logits - m)
    s = jnp.sum(ex, axis=-1, keepdims=True)
    probs = ex / s
    lse = m + jnp.log(s)

    e = jnp.argmax(logits, axis=-1).astype(jnp.int32)
    eids = lax.broadcasted_iota(jnp.int32, (BLK, NUM_EXPERTS), 1)
    onehot = (eids == e[:, None]).astype(jnp.float32)

    r_i = lax.broadcasted_iota(jnp.int32, (BLK, BLK), 0)
    c_i = lax.broadcasted_iota(jnp.int32, (BLK, BLK), 1)
    tril = (c_i < r_i).astype(jnp.float32)
    excl = lax.dot_general(tril, onehot, (((1,), (0,)), ((), ())),
                           preferred_element_type=jnp.float32)

    @pl.when(i == 0)
    def _():
        counts_ref[...] = jnp.zeros_like(counts_ref)
        sump_ref[...] = jnp.zeros_like(sump_ref)
        hz_ref[...] = jnp.zeros_like(hz_ref)

    carry = counts_ref[0, :NUM_EXPERTS]
    rank = jnp.sum(onehot * (excl + carry[None, :]), axis=-1)
    e_ref[...] = e
    rank_ref[...] = rank.astype(jnp.int32)

    blk_counts = jnp.sum(onehot, axis=0)
    counts_ref[0, :NUM_EXPERTS] += blk_counts
    sump_ref[0, :NUM_EXPERTS] += jnp.sum(probs, axis=0)
    token_h = -jnp.sum(probs * jnp.log(probs + 1e-9), axis=-1)
    lane = lax.broadcasted_iota(jnp.int32, (1, 16), 1)
    hz_ref[...] += (jnp.where(lane == 0, jnp.sum(token_h), 0.0)
                    + jnp.where(lane == 1, jnp.sum(lse[:, 0] ** 2), 0.0))


def _router(x_flat, router_w):
    return pl.pallas_call(
        _router_body,
        grid=(RB,),
        in_specs=[
            pl.BlockSpec((BLK, N_EMBD), lambda i: (i, 0)),
            pl.BlockSpec((NUM_EXPERTS, N_EMBD), lambda i: (0, 0)),
        ],
        out_specs=[
            pl.BlockSpec((BLK,), lambda i: (i,)),
            pl.BlockSpec((BLK,), lambda i: (i,)),
            pl.BlockSpec((1, 16), lambda i: (0, 0)),
            pl.BlockSpec((1, 16), lambda i: (0, 0)),
            pl.BlockSpec((1, 16), lambda i: (0, 0)),
        ],
        out_shape=[
            jax.ShapeDtypeStruct((BT,), jnp.int32),
            jax.ShapeDtypeStruct((BT,), jnp.int32),
            jax.ShapeDtypeStruct((1, 16), jnp.float32),
            jax.ShapeDtypeStruct((1, 16), jnp.float32),
            jax.ShapeDtypeStruct((1, 16), jnp.float32),
        ],
    )(x_flat, router_w)


def _dest_body(po_ref, e_ref, rank_ref, dest_ref):
    e = e_ref[...]
    dest = rank_ref[...]
    for k in range(NUM_EXPERTS):
        dest = dest + jnp.where(e == k, po_ref[k], 0)
    dest_ref[...] = dest


def _dest(po16, e, rank):
    return pl.pallas_call(
        _dest_body,
        grid=(RB,),
        in_specs=[
            pl.BlockSpec(memory_space=pltpu.SMEM),
            pl.BlockSpec((BLK,), lambda i: (i,)),
            pl.BlockSpec((BLK,), lambda i: (i,)),
        ],
        out_specs=pl.BlockSpec((BLK,), lambda i: (i,)),
        out_shape=jax.ShapeDtypeStruct((BT,), jnp.int32),
    )(po16, e, rank)


_NW = 32
_TPW = BT // _NW
_ROWS = 16
_NCH = _TPW // _ROWS


def _fill_idx(idx_v, dest_v, c):
    idx_v[...] = dest_v[pl.ds(c * _ROWS, 16)]


def _dispatch_body(dest_hbm, x_hbm, xs_hbm, dest_v,
                   idx0, idx1, r0, r1, sr0, sr1, sw0, sw1):
    wid = lax.axis_index("s") * 2 + lax.axis_index("c")
    base = wid * _TPW
    pltpu.sync_copy(dest_hbm.at[pl.ds(base, _TPW)], dest_v)
    idxb, rb = (idx0, idx1), (r0, r1)
    srs, sws = (sr0, sr1), (sw0, sw1)
    for c in range(_NCH):
        cur = c & 1
        if c >= 2:
            pltpu.make_async_copy(rb[cur], xs_hbm.at[idxb[cur]],
                                  sws[cur]).wait()
        _fill_idx(idxb[cur], dest_v, c)
        rd = pltpu.make_async_copy(x_hbm.at[pl.ds(base + c * _ROWS, _ROWS)],
                                   rb[cur], srs[cur])
        rd.start()
        rd.wait()
        pltpu.make_async_copy(rb[cur], xs_hbm.at[idxb[cur]], sws[cur]).start()
    for c in range(max(_NCH - 2, 0), _NCH):
        cur = c & 1
        pltpu.make_async_copy(rb[cur], xs_hbm.at[idxb[cur]], sws[cur]).wait()


def _combine_body(dest_hbm, ys_hbm, y_hbm, dest_v,
                  idx0, idx1, r0, r1, sr0, sr1, sw0, sw1):
    wid = lax.axis_index("s") * 2 + lax.axis_index("c")
    base = wid * _TPW
    pltpu.sync_copy(dest_hbm.at[pl.ds(base, _TPW)], dest_v)
    idxb, rb = (idx0, idx1), (r0, r1)
    srs, sws = (sr0, sr1), (sw0, sw1)
    for c in range(_NCH):
        cur = c & 1
        if c >= 2:
            pltpu.make_async_copy(
                rb[cur], y_hbm.at[pl.ds(base + (c - 2) * _ROWS, _ROWS)],
                sws[cur]).wait()
        _fill_idx(idxb[cur], dest_v, c)
        rd = pltpu.make_async_copy(ys_hbm.at[idxb[cur]], rb[cur], srs[cur])
        rd.start()
        rd.wait()
        pltpu.make_async_copy(rb[cur],
                              y_hbm.at[pl.ds(base + c * _ROWS, _ROWS)],
                              sws[cur]).start()
    for c in range(max(_NCH - 2, 0), _NCH):
        cur = c & 1
        pltpu.make_async_copy(rb[cur],
                              y_hbm.at[pl.ds(base + c * _ROWS, _ROWS)],
                              sws[cur]).wait()


_SC_SCRATCH = [
    pltpu.VMEM((_TPW,), jnp.int32),
    pltpu.VMEM((_ROWS,), jnp.int32),
    pltpu.VMEM((_ROWS,), jnp.int32),
    pltpu.VMEM((_ROWS, N_EMBD), jnp.float32),
    pltpu.VMEM((_ROWS, N_EMBD), jnp.float32),
    pltpu.SemaphoreType.DMA,
    pltpu.SemaphoreType.DMA,
    pltpu.SemaphoreType.DMA,
    pltpu.SemaphoreType.DMA,
]


@functools.lru_cache(maxsize=None)
def _sc_kernels():
    mesh = plsc.VectorSubcoreMesh(core_axis_name="c", subcore_axis_name="s",
                                  num_cores=2, num_subcores=16)
    dispatch = pl.kernel(
        _dispatch_body,
        out_type=jax.ShapeDtypeStruct((NPAD, N_EMBD), jnp.float32),
        mesh=mesh,
        scratch_types=list(_SC_SCRATCH),
    )
    combine = pl.kernel(
        _combine_body,
        out_type=jax.ShapeDtypeStruct((BT, N_EMBD), jnp.float32),
        mesh=mesh,
        scratch_types=list(_SC_SCRATCH),
    )
    return dispatch, combine


def _mlp_body(meta_ref, xs_ref, wfc_ref, wpj_ref, yin_ref, out_ref):
    del yin_ref
    b = pl.program_id(0)
    h = pl.program_id(1)
    nb = meta_ref[1]

    @pl.when(b < nb)
    def _():
        x = xs_ref[...].astype(jnp.bfloat16)
        wfc = wfc_ref[...].astype(jnp.bfloat16)
        wpj = wpj_ref[...].astype(jnp.bfloat16)
        hact = lax.dot_general(x, wfc, (((1,), (1,)), ((), ())),
                               preferred_element_type=jnp.float32)
        hact = jnp.square(jnp.maximum(hact, 0.0)).astype(jnp.bfloat16)
        part = lax.dot_general(hact, wpj, (((1,), (1,)), ((), ())),
                               preferred_element_type=jnp.float32)

        @pl.when(h == 0)
        def _():
            out_ref[...] = part

        @pl.when(h != 0)
        def _():
            out_ref[...] += part


def _tok_idx(b, h, meta):
    return (meta[0] + jnp.minimum(b, meta[1] - 1), 0)


def _zig_idx(b, h, meta):
    nb = meta[1]
    zz = jnp.where(b % 2 == 0, h, H_TILES - 1 - h)
    frozen = jnp.where((nb - 1) % 2 == 0, H_TILES - 1, 0)
    return jnp.where(b < nb, zz, frozen)


def _mlp_body_first(meta_ref, xs_ref, wfc_ref, wpj_ref, out_ref):
    _mlp_body(meta_ref, xs_ref, wfc_ref, wpj_ref, None, out_ref)


def _expert_mlp(meta, xs, wfc, wpj, ys=None):
    in_specs = [
        pl.BlockSpec((BLK, N_EMBD), _tok_idx),
        pl.BlockSpec((H_CHUNK, N_EMBD),
                     lambda b, h, meta: (_zig_idx(b, h, meta), 0)),
        pl.BlockSpec((N_EMBD, H_CHUNK),
                     lambda b, h, meta: (0, _zig_idx(b, h, meta))),
    ]
    args = (meta, xs, wfc, wpj)
    if ys is None:
        body, aliases = _mlp_body_first, {}
    else:
        in_specs.append(pl.BlockSpec((8, 128), lambda b, h, meta: (0, 0)))
        args += (ys,)
        body, aliases = _mlp_body, {4: 0}
    grid_spec = pltpu.PrefetchScalarGridSpec(
        num_scalar_prefetch=1,
        grid=(RB, H_TILES),
        in_specs=in_specs,
        out_specs=pl.BlockSpec((BLK, N_EMBD), _tok_idx),
    )
    return pl.pallas_call(
        body,
        grid_spec=grid_spec,
        out_shape=jax.ShapeDtypeStruct((NPAD, N_EMBD), jnp.float32),
        input_output_aliases=aliases,
        compiler_params=pltpu.CompilerParams(
            dimension_semantics=("arbitrary", "arbitrary"),
        ),
    )(*args)


def kernel(x, router_w, w_fc_0, w_proj_0, w_fc_1, w_proj_1,
           w_fc_2, w_proj_2, w_fc_3, w_proj_3):
    B, T, C = x.shape
    x_flat = x.reshape(BT, C)

    e, rank, counts_f, sump, hz = _router(x_flat, router_w)
    counts = counts_f[0, :NUM_EXPERTS]
    counts_i = counts.astype(jnp.int32)

    nbb = jnp.maximum((counts_i + (BLK - 1)) // BLK, 1)
    pob = jnp.concatenate(
        [jnp.zeros((1,), jnp.int32), jnp.cumsum(nbb).astype(jnp.int32)])
    po16 = jnp.zeros((16,), jnp.int32).at[:NUM_EXPERTS].set(pob[:NUM_EXPERTS]
                                                            * BLK)
    dest = _dest(po16, e, rank)
    dispatch, combine = _sc_kernels()
    xs = dispatch(dest, x_flat)
    ys = None
    for ei, (wfc, wpj) in enumerate([(w_fc_0, w_proj_0), (w_fc_1, w_proj_1),
                                     (w_fc_2, w_proj_2), (w_fc_3, w_proj_3)]):
        meta = jnp.stack([pob[ei], nbb[ei]])
        ys = _expert_mlp(meta, xs, wfc, wpj, ys)
    y_flat = combine(dest, ys)

    inv_bt = 1.0 / float(BT)
    actual = counts * inv_bt
    expected = sump[0, :NUM_EXPERTS] * inv_bt
    aux = NUM_EXPERTS * jnp.sum(actual * expected)
    router_entropy = hz[0, 0] * inv_bt / math.log(float(NUM_EXPERTS))
    z_loss = hz[0, 1] * inv_bt
    return (y_flat.reshape(B, T, C), aux, z_loss, router_entropy, actual)
